# baseline (speedup 1.0000x reference)
import functools

import jax
import jax.numpy as jnp
from jax import lax
from jax.experimental import pallas as pl
from jax.experimental.pallas import tpu as pltpu
from jax.experimental.pallas import tpu_sc as plsc

N = 10000
E = 320000
D_IN = 128
H = 240
S0 = 64
L = 3
NG = 64
LATENT = 128

HP = 256
NP_ = 10240
EP = 327680
CHUNK = 128
NCORE = 2
NSUB = 16
NW = NCORE * NSUB


def _pipeline(nch, nslots, issue1, wait1, issue2, wait2):
    fire = nslots - 2
    for b in range(fire):
        issue1(b)

    @pl.loop(0, nch)
    def _(c):
        wait1()
        issue2(c)

        @pl.when(c >= 2)
        def _():
            wait2()

        @pl.when(c + fire < nch)
        def _():
            issue1(c + fire)

    wait2()
    wait2()


def _make_sc_gather(width, rpc, nslots, nrows):
    mesh = plsc.VectorSubcoreMesh(core_axis_name="c", subcore_axis_name="s")
    ct = nrows // rpc
    ipr = CHUNK // rpc
    aln = 8 * ipr
    a0 = ((4 * ct // 5) // NSUB) // aln * aln
    a1 = (ct - a0 * NSUB) // NSUB
    assert a0 % aln == 0 and a1 % aln == 0 and a1 >= nslots - 2
    ir = a0 // ipr

    def make(npairs):
        @functools.partial(
            pl.kernel, mesh=mesh,
            out_type=[jax.ShapeDtypeStruct((EP, width), jnp.float32)] * npairs,
            scratch_types=[
                pltpu.VMEM((ir, CHUNK), jnp.int32),
                pltpu.VMEM((nslots * rpc, width), jnp.float32),
                pltpu.SemaphoreType.DMA,
                pltpu.SemaphoreType.DMA,
            ],
        )
        def k(table, *rest):
            idxs = rest[:npairs]
            outs = rest[npairs:2 * npairs]
            idx_v, rows_v, sem1, sem2 = rest[2 * npairs:]
            c = lax.axis_index("c")
            s = lax.axis_index("s")
            nch = jnp.where(c == 0, a0, a1)
            cbase = jnp.where(c == 0, s * a0, a0 * NSUB + s * a1)

            def slot(ch):
                return rows_v.at[pl.ds(lax.rem(ch, nslots) * rpc, rpc)]

            for idx, out in zip(idxs, outs):
                pltpu.sync_copy(idx.at[pl.ds(lax.div(cbase, ipr), ir)], idx_v)

                def issue1(ch):
                    iv = (idx_v.at[ch, :] if ipr == 1 else
                          idx_v.at[lax.div(ch, ipr),
                                   pl.ds(lax.rem(ch, ipr) * rpc, rpc)])
                    pltpu.async_copy(table.at[iv], slot(ch), sem1)

                def wait1():
                    pltpu.make_async_copy(
                        table.at[pl.ds(0, rpc)], rows_v.at[pl.ds(0, rpc)],
                        sem1).wait()

                def issue2(ch):
                    pltpu.async_copy(
                        slot(ch),
                        out.at[pl.ds((cbase + ch) * rpc, rpc)], sem2)

                def wait2():
                    pltpu.make_async_copy(
                        rows_v.at[pl.ds(0, rpc)], out.at[pl.ds(0, rpc)],
                        sem2).wait()

                _pipeline(nch, nslots, issue1, wait1, issue2, wait2)

        return k

    return make


def _make_sc_scatter_add(nslots):
    mesh = plsc.VectorSubcoreMesh(core_axis_name="c", subcore_axis_name="s")
    cps = (EP // NSUB) // CHUNK
    NPA = 10112
    rps = NPA // NSUB

    @functools.partial(
        pl.kernel, mesh=mesh,
        out_type=jax.ShapeDtypeStruct((2, NP_, 128), jnp.float32),
        scratch_types=[
            pltpu.VMEM((nslots, CHUNK), jnp.int32),
            pltpu.VMEM((nslots * CHUNK, 128), jnp.float32),
            pltpu.VMEM_SHARED((NPA, 128), jnp.float32),
            pltpu.SemaphoreType.DMA,
            pltpu.SemaphoreType.DMA,
        ],
    )
    def k(gatedA, gatedB, idx, agg2, idx_v, dat_v, acc_sh, sem1, sem2):
        c = lax.axis_index("c")
        s = lax.axis_index("s")

        zv = jnp.zeros((16,), jnp.float32)

        @pl.loop(0, CHUNK)
        def _(r):
            for seg in range(8):
                dat_v[r, pl.ds(seg * 16, 16)] = zv

        done = 0
        while done < rps:
            step = min(CHUNK, rps - done)
            pltpu.sync_copy(dat_v.at[pl.ds(0, step)],
                            acc_sh.at[pl.ds(s * rps + done, step)])
            done += step

        @pl.when(s == 0)
        def _():
            pltpu.sync_copy(dat_v.at[pl.ds(0, NP_ - NPA)],
                            agg2.at[c, pl.ds(NPA, NP_ - NPA), :])
        plsc.subcore_barrier()

        def slot(ch):
            return dat_v.at[pl.ds(lax.rem(ch, nslots) * CHUNK, CHUNK)]

        def islot(ch):
            return idx_v.at[lax.rem(ch, nslots)]

        def issue1(ch):
            @pl.when(s < NSUB // 2)
            def _():
                pltpu.async_copy(
                    gatedA.at[c, pl.ds((s * cps + ch) * CHUNK, CHUNK), :],
                    slot(ch), sem1)

            @pl.when(s >= NSUB // 2)
            def _():
                pltpu.async_copy(
                    gatedB.at[c, pl.ds(((s - NSUB // 2) * cps + ch) * CHUNK,
                                       CHUNK), :],
                    slot(ch), sem1)
            pltpu.async_copy(idx.at[s * cps + ch], islot(ch), sem1)

        def wait1():
            pltpu.make_async_copy(
                gatedA.at[0, pl.ds(0, CHUNK), :],
                dat_v.at[pl.ds(0, CHUNK)], sem1).wait()
            pltpu.make_async_copy(idx.at[0], idx_v.at[0], sem1).wait()

        def issue2(ch):
            pltpu.async_copy(slot(ch), acc_sh.at[islot(ch)],
                             sem2, add=True)

        def wait2():
            pltpu.make_async_copy(
                dat_v.at[pl.ds(0, CHUNK)], acc_sh.at[pl.ds(0, CHUNK)],
                sem2).wait()

        _pipeline(cps, nslots, issue1, wait1, issue2, wait2)
        plsc.subcore_barrier()
        pltpu.sync_copy(acc_sh.at[pl.ds(s * rps, rps)],
                        agg2.at[c, pl.ds(s * rps, rps), :])

    return k


EPH = EP // 2

_sc_gather_cached = functools.lru_cache(None)(
    lambda width, rpc, nslots, nrows, npairs:
        _make_sc_gather(width, rpc, nslots, nrows)(npairs))
_sc_scatter_cached = functools.lru_cache(None)(_make_sc_scatter_add)


def _sc_gather_pos(table, idx_a, idx_b):
    return _sc_gather_cached(128, 128, 6, EPH, 2)(table, idx_a, idx_b)


def _sc_gather256(table, idx):
    res = _sc_gather_cached(HP, 64, 6, EPH, 1)(table, idx)
    return res[0] if isinstance(res, (list, tuple)) else res


def _sc_scatter(gatedA, gatedB, idx):
    return _sc_scatter_cached(3)(gatedA, gatedB, idx)


_BLK_N = 1024
_BLK_E = 1024


def _edge_attr_call(pr, pc):
    def body(pr_ref, pc_ref, out_ref):
        r = pr_ref[...] - pc_ref[...]
        d2 = jnp.sum(r * r, axis=1, keepdims=True)
        dist = jnp.sqrt(d2)
        inv = 1.0 / (dist + 1e-8)
        lane = lax.broadcasted_iota(jnp.int32, (_BLK_E, 16), 1)
        out_ref[...] = r[:, :16] * inv + jnp.where(lane == 3, dist, 0.0)

    ne = pr.shape[0]
    return pl.pallas_call(
        body,
        grid=(ne // _BLK_E,),
        in_specs=[pl.BlockSpec((_BLK_E, 128), lambda i: (i, 0)),
                  pl.BlockSpec((_BLK_E, 128), lambda i: (i, 0))],
        out_specs=pl.BlockSpec((_BLK_E, 16), lambda i: (i, 0)),
        out_shape=jax.ShapeDtypeStruct((ne, 16), jnp.float32),
    )(pr, pc)


def _prologue_call(xp, w_in, wn0p):
    def body(x_ref, win_ref, wn_ref, h_ref, g_ref):
        h = jnp.dot(x_ref[...], win_ref[...], preferred_element_type=jnp.float32)
        h_ref[...] = h
        g_ref[...] = jnp.dot(h, wn_ref[...], preferred_element_type=jnp.float32)

    return pl.pallas_call(
        body,
        grid=(NP_ // _BLK_N,),
        in_specs=[pl.BlockSpec((_BLK_N, D_IN), lambda i: (i, 0)),
                  pl.BlockSpec((D_IN, H), lambda i: (0, 0)),
                  pl.BlockSpec((H, HP), lambda i: (0, 0))],
        out_specs=[pl.BlockSpec((_BLK_N, H), lambda i: (i, 0)),
                   pl.BlockSpec((_BLK_N, HP), lambda i: (i, 0))],
        out_shape=[jax.ShapeDtypeStruct((NP_, H), jnp.float32),
                   jax.ShapeDtypeStruct((NP_, HP), jnp.float32)],
    )(xp, w_in, wn0p)


def _gate_call(gath, ea, wep, wgext):
    def body(g_ref, ea_ref, we_ref, wg_ref, out_ref):
        a = jnp.dot(ea_ref[...], we_ref[...], preferred_element_type=jnp.float32)
        m = g_ref[...] * a
        z = jnp.dot(m[:, :S0], wg_ref[...], preferred_element_type=jnp.float32)
        res = m * jax.nn.sigmoid(z)
        out_ref[0] = res[:, :128]
        out_ref[1] = res[:, 128:]

    ne = gath.shape[0]
    return pl.pallas_call(
        body,
        grid=(ne // _BLK_E,),
        in_specs=[pl.BlockSpec((_BLK_E, HP), lambda i: (i, 0)),
                  pl.BlockSpec((_BLK_E, 16), lambda i: (i, 0)),
                  pl.BlockSpec((16, HP), lambda i: (0, 0)),
                  pl.BlockSpec((S0, HP), lambda i: (0, 0))],
        out_specs=pl.BlockSpec((2, _BLK_E, 128), lambda i: (0, i, 0)),
        out_shape=jax.ShapeDtypeStruct((2, ne, 128), jnp.float32),
    )(gath, ea, wep, wgext)


def _update_call(h, agg, wlin_msg, wn_upd, wm_upd, wg_ext, wlin_upd, wn_next):
    has_next = wn_next is not None

    def body(h_ref, agg_ref, wl_ref, wn_ref, wm_ref, wg_ref, wu_ref,
             *rest):
        if has_next:
            wnx_ref, h_out, g_out = rest
        else:
            (h_out,) = rest
        h_ = h_ref[...]
        agg = jnp.concatenate([agg_ref[0], agg_ref[1][:, :H - 128]], axis=1)
        aggw = jnp.dot(agg, wl_ref[...],
                       preferred_element_type=jnp.float32)
        u1 = (jnp.dot(h_, wn_ref[...], preferred_element_type=jnp.float32)
              * jnp.dot(aggw, wm_ref[...], preferred_element_type=jnp.float32))
        z = jnp.dot(u1[:, :S0], wg_ref[...], preferred_element_type=jnp.float32)
        u2 = u1 * jax.nn.sigmoid(z)
        hn = h_ + jnp.dot(u2, wu_ref[...], preferred_element_type=jnp.float32)
        h_out[...] = hn
        if has_next:
            g_out[...] = jnp.dot(hn, wnx_ref[...],
                                 preferred_element_type=jnp.float32)

    in_specs = [pl.BlockSpec((_BLK_N, H), lambda i: (i, 0)),
                pl.BlockSpec((2, _BLK_N, 128), lambda i: (0, i, 0)),
                pl.BlockSpec((H, H), lambda i: (0, 0)),
                pl.BlockSpec((H, H), lambda i: (0, 0)),
                pl.BlockSpec((H, H), lambda i: (0, 0)),
                pl.BlockSpec((S0, H), lambda i: (0, 0)),
                pl.BlockSpec((H, H), lambda i: (0, 0))]
    out_specs = [pl.BlockSpec((_BLK_N, H), lambda i: (i, 0))]
    out_shape = [jax.ShapeDtypeStruct((NP_, H), jnp.float32)]
    args = [h, agg, wlin_msg, wn_upd, wm_upd, wg_ext, wlin_upd]
    if has_next:
        in_specs.append(pl.BlockSpec((H, HP), lambda i: (0, 0)))
        out_specs.append(pl.BlockSpec((_BLK_N, HP), lambda i: (i, 0)))
        out_shape.append(jax.ShapeDtypeStruct((NP_, HP), jnp.float32))
        args.append(wn_next)

    return pl.pallas_call(
        body,
        grid=(NP_ // _BLK_N,),
        in_specs=in_specs,
        out_specs=out_specs,
        out_shape=out_shape,
    )(*args)


def _pool_call(h, batch2, w_final):
    nb = NP_ // _BLK_N

    def body(h_ref, b_ref, wf_ref, out_ref, acc_s, cnt_s):
        i = pl.program_id(0)

        @pl.when(i == 0)
        def _():
            acc_s[...] = jnp.zeros_like(acc_s)
            cnt_s[...] = jnp.zeros_like(cnt_s)

        inv = h_ref[:, :S0]
        b = b_ref[...]
        oh = (b == lax.broadcasted_iota(jnp.int32, (_BLK_N, NG), 1)
              ).astype(jnp.float32)
        acc_s[...] += lax.dot_general(oh, inv, (((0,), (0,)), ((), ())),
                                      preferred_element_type=jnp.float32)
        cnt_s[...] += lax.dot_general(oh, jnp.ones((_BLK_N, 8), jnp.float32),
                                      (((0,), (0,)), ((), ())),
                                      preferred_element_type=jnp.float32)

        @pl.when(i == nb - 1)
        def _():
            pooled = acc_s[...] / jnp.maximum(cnt_s[:, :1], 1.0)
            out_ref[...] = jnp.dot(pooled, wf_ref[...],
                                   preferred_element_type=jnp.float32)

    return pl.pallas_call(
        body,
        grid=(nb,),
        in_specs=[pl.BlockSpec((_BLK_N, H), lambda i: (i, 0)),
                  pl.BlockSpec((_BLK_N, 1), lambda i: (i, 0)),
                  pl.BlockSpec((S0, LATENT), lambda i: (0, 0))],
        out_specs=pl.BlockSpec((NG, LATENT), lambda i: (0, 0)),
        out_shape=jax.ShapeDtypeStruct((NG, LATENT), jnp.float32),
        scratch_shapes=[pltpu.VMEM((NG, S0), jnp.float32),
                        pltpu.VMEM((NG, 8), jnp.float32)],
    )(h, batch2, w_final)


@jax.jit
def kernel(x, pos, edge_index, batch, W_in, Wn_msg, We_msg, Wg_msg, Wlin_msg,
           Wn_upd, Wm_upd, Wg_upd, Wlin_upd, W_final):
    f32 = jnp.float32
    row = edge_index[0]
    col = edge_index[1]

    xp = jnp.zeros((NP_, D_IN), f32).at[:N].set(x)
    pos128 = jnp.zeros((NP_, 128), f32).at[:N, :3].set(pos)
    idx_pad = jnp.zeros((64, CHUNK), jnp.int32)
    rowp = jnp.concatenate(
        [row, jnp.full((EP - E,), N, jnp.int32)]).reshape(EP // CHUNK, CHUNK)
    colp = jnp.concatenate(
        [col, jnp.zeros((EP - E,), jnp.int32)]).reshape(EP // CHUNK, CHUNK)
    hr = EPH // CHUNK
    rowgA = jnp.concatenate([rowp[:hr], idx_pad], axis=0)
    rowgB = jnp.concatenate([rowp[hr:], idx_pad], axis=0)
    colgA = jnp.concatenate([colp[:hr], idx_pad], axis=0)
    colgB = jnp.concatenate([colp[hr:], idx_pad], axis=0)
    batch2 = jnp.concatenate(
        [batch, jnp.full((NP_ - N,), NG, jnp.int32)]).reshape(NP_, 1)

    eye = jnp.eye(S0, dtype=f32)
    wn_msg_p = jnp.zeros((L, H, HP), f32).at[:, :, :H].set(Wn_msg)
    we_p = jnp.zeros((L, 16, HP), f32).at[:, :4, :H].set(We_msg)
    wg_msg_ext = jnp.zeros((L, S0, HP), f32)
    wg_msg_ext = wg_msg_ext.at[:, :, :S0].set(eye)
    wg_msg_ext = wg_msg_ext.at[:, :, S0:H].set(Wg_msg)
    wg_upd_ext = jnp.concatenate(
        [jnp.broadcast_to(eye, (L, S0, S0)), Wg_upd], axis=2)

    prA, pcA = _sc_gather_pos(pos128, rowgA, colgA)
    eaA = _edge_attr_call(prA, pcA)
    prB, pcB = _sc_gather_pos(pos128, rowgB, colgB)
    eaB = _edge_attr_call(prB, pcB)

    h, g = _prologue_call(xp, W_in, wn_msg_p[0])
    for l in range(L):
        gathA = _sc_gather256(g, colgA)
        gatedA = _gate_call(gathA, eaA, we_p[l], wg_msg_ext[l])
        gathB = _sc_gather256(g, colgB)
        gatedB = _gate_call(gathB, eaB, we_p[l], wg_msg_ext[l])
        agg = _sc_scatter(gatedA, gatedB, rowp)
        wn_next = wn_msg_p[l + 1] if l + 1 < L else None
        res = _update_call(h, agg, Wlin_msg[l], Wn_upd[l], Wm_upd[l],
                           wg_upd_ext[l], Wlin_upd[l], wn_next)
        if wn_next is not None:
            h, g = res
        else:
            (h,) = res

    return _pool_call(h, batch2, W_final)


# speedup vs baseline: 1.2036x; 1.2036x over previous
import functools

import jax
import jax.numpy as jnp
from jax import lax
from jax.experimental import pallas as pl
from jax.experimental.pallas import tpu as pltpu
from jax.experimental.pallas import tpu_sc as plsc

N = 10000
E = 320000
D_IN = 128
H = 240
S0 = 64
L = 3
NG = 64
LATENT = 128

HP = 256
NP_ = 10240
EP = 327680
CHUNK = 128
NCORE = 2
NSUB = 16
NW = NCORE * NSUB


def _pipeline(nch, nslots, issue1, wait1, issue2, wait2):
    fire = nslots - 2
    for b in range(fire):
        issue1(b)

    @pl.loop(0, nch)
    def _(c):
        wait1()
        issue2(c)

        @pl.when(c >= 2)
        def _():
            wait2()

        @pl.when(c + fire < nch)
        def _():
            issue1(c + fire)

    wait2()
    wait2()


def _make_sc_gather(width, rpc, nslots, nrows):
    mesh = plsc.VectorSubcoreMesh(core_axis_name="c", subcore_axis_name="s")
    ct = nrows // rpc
    ipr = CHUNK // rpc
    aln = 8 * ipr
    a0 = ((3 * ct // 4) // NSUB) // aln * aln
    a1 = (ct - a0 * NSUB) // NSUB
    assert a0 % aln == 0 and a1 % aln == 0 and a1 >= nslots - 2
    ir = a0 // ipr

    def make(npairs):
        @functools.partial(
            pl.kernel, mesh=mesh,
            out_type=[jax.ShapeDtypeStruct((EP, width), jnp.float32)] * npairs,
            scratch_types=[
                pltpu.VMEM((ir, CHUNK), jnp.int32),
                pltpu.VMEM((nslots * rpc, width), jnp.float32),
                pltpu.SemaphoreType.DMA,
                pltpu.SemaphoreType.DMA,
            ],
        )
        def k(table, *rest):
            idxs = rest[:npairs]
            outs = rest[npairs:2 * npairs]
            idx_v, rows_v, sem1, sem2 = rest[2 * npairs:]
            c = lax.axis_index("c")
            s = lax.axis_index("s")
            nch = jnp.where(c == 0, a0, a1)
            cbase = jnp.where(c == 0, s * a0, a0 * NSUB + s * a1)

            def slot(ch):
                return rows_v.at[pl.ds(lax.rem(ch, nslots) * rpc, rpc)]

            for idx, out in zip(idxs, outs):
                pltpu.sync_copy(idx.at[pl.ds(lax.div(cbase, ipr), ir)], idx_v)

                def issue1(ch):
                    iv = (idx_v.at[ch, :] if ipr == 1 else
                          idx_v.at[lax.div(ch, ipr),
                                   pl.ds(lax.rem(ch, ipr) * rpc, rpc)])
                    pltpu.async_copy(table.at[iv], slot(ch), sem1)

                def wait1():
                    pltpu.make_async_copy(
                        table.at[pl.ds(0, rpc)], rows_v.at[pl.ds(0, rpc)],
                        sem1).wait()

                def issue2(ch):
                    pltpu.async_copy(
                        slot(ch),
                        out.at[pl.ds((cbase + ch) * rpc, rpc)], sem2)

                def wait2():
                    pltpu.make_async_copy(
                        rows_v.at[pl.ds(0, rpc)], out.at[pl.ds(0, rpc)],
                        sem2).wait()

                _pipeline(nch, nslots, issue1, wait1, issue2, wait2)

        return k

    return make


def _make_sc_scatter_add(nslots):
    mesh = plsc.VectorSubcoreMesh(core_axis_name="c", subcore_axis_name="s")
    cps = (EP // NSUB) // CHUNK
    NPA = 10112
    rps = NPA // NSUB

    @functools.partial(
        pl.kernel, mesh=mesh,
        out_type=jax.ShapeDtypeStruct((2, NP_, 128), jnp.float32),
        scratch_types=[
            pltpu.VMEM((nslots, CHUNK), jnp.int32),
            pltpu.VMEM((nslots * CHUNK, 128), jnp.float32),
            pltpu.VMEM_SHARED((NPA, 128), jnp.float32),
            pltpu.SemaphoreType.DMA,
            pltpu.SemaphoreType.DMA,
        ],
    )
    def k(gated2, idx, agg2, idx_v, dat_v, acc_sh, sem1, sem2):
        c = lax.axis_index("c")
        s = lax.axis_index("s")

        zv = jnp.zeros((16,), jnp.float32)

        @pl.loop(0, CHUNK)
        def _(r):
            for seg in range(8):
                dat_v[r, pl.ds(seg * 16, 16)] = zv

        done = 0
        while done < rps:
            step = min(CHUNK, rps - done)
            pltpu.sync_copy(dat_v.at[pl.ds(0, step)],
                            acc_sh.at[pl.ds(s * rps + done, step)])
            done += step

        @pl.when(s == 0)
        def _():
            pltpu.sync_copy(dat_v.at[pl.ds(0, NP_ - NPA)],
                            agg2.at[c, pl.ds(NPA, NP_ - NPA), :])
        plsc.subcore_barrier()

        def slot(ch):
            return dat_v.at[pl.ds(lax.rem(ch, nslots) * CHUNK, CHUNK)]

        def islot(ch):
            return idx_v.at[lax.rem(ch, nslots)]

        def issue1(ch):
            pltpu.async_copy(
                gated2.at[c, pl.ds((s * cps + ch) * CHUNK, CHUNK), :],
                slot(ch), sem1)
            pltpu.async_copy(idx.at[s * cps + ch], islot(ch), sem1)

        def wait1():
            pltpu.make_async_copy(
                gated2.at[0, pl.ds(0, CHUNK), :],
                dat_v.at[pl.ds(0, CHUNK)], sem1).wait()
            pltpu.make_async_copy(idx.at[0], idx_v.at[0], sem1).wait()

        def issue2(ch):
            pltpu.async_copy(slot(ch), acc_sh.at[islot(ch)],
                             sem2, add=True)

        def wait2():
            pltpu.make_async_copy(
                dat_v.at[pl.ds(0, CHUNK)], acc_sh.at[pl.ds(0, CHUNK)],
                sem2).wait()

        _pipeline(cps, nslots, issue1, wait1, issue2, wait2)
        plsc.subcore_barrier()
        pltpu.sync_copy(acc_sh.at[pl.ds(s * rps, rps)],
                        agg2.at[c, pl.ds(s * rps, rps), :])

    return k


EPH = EP // 2

_sc_gather_cached = functools.lru_cache(None)(
    lambda width, rpc, nslots, nrows, npairs:
        _make_sc_gather(width, rpc, nslots, nrows)(npairs))
_sc_scatter_cached = functools.lru_cache(None)(_make_sc_scatter_add)


def _sc_gather_pos(table, idx_a, idx_b):
    return _sc_gather_cached(128, 128, 6, EP, 2)(table, idx_a, idx_b)


def _sc_gather256(table, idx):
    res = _sc_gather_cached(HP, 64, 7, EP, 1)(table, idx)
    return res[0] if isinstance(res, (list, tuple)) else res


def _sc_scatter(gated2, idx):
    return _sc_scatter_cached(3)(gated2, idx)


_BLK_N = 1024
_BLK_E = 1024


def _edge_attr_call(pr, pc):
    def body(pr_ref, pc_ref, out_ref):
        r = pr_ref[...] - pc_ref[...]
        d2 = jnp.sum(r * r, axis=1, keepdims=True)
        dist = jnp.sqrt(d2)
        inv = 1.0 / (dist + 1e-8)
        lane = lax.broadcasted_iota(jnp.int32, (_BLK_E, 16), 1)
        out_ref[...] = r[:, :16] * inv + jnp.where(lane == 3, dist, 0.0)

    ne = pr.shape[0]
    return pl.pallas_call(
        body,
        grid=(ne // _BLK_E,),
        in_specs=[pl.BlockSpec((_BLK_E, 128), lambda i: (i, 0)),
                  pl.BlockSpec((_BLK_E, 128), lambda i: (i, 0))],
        out_specs=pl.BlockSpec((_BLK_E, 16), lambda i: (i, 0)),
        out_shape=jax.ShapeDtypeStruct((ne, 16), jnp.float32),
    )(pr, pc)


def _prologue_call(xp, w_in, wn0p):
    def body(x_ref, win_ref, wn_ref, h_ref, g_ref):
        h = jnp.dot(x_ref[...], win_ref[...], preferred_element_type=jnp.float32)
        h_ref[...] = h
        g_ref[...] = jnp.dot(h, wn_ref[...], preferred_element_type=jnp.float32)

    return pl.pallas_call(
        body,
        grid=(NP_ // _BLK_N,),
        in_specs=[pl.BlockSpec((_BLK_N, D_IN), lambda i: (i, 0)),
                  pl.BlockSpec((D_IN, H), lambda i: (0, 0)),
                  pl.BlockSpec((H, HP), lambda i: (0, 0))],
        out_specs=[pl.BlockSpec((_BLK_N, H), lambda i: (i, 0)),
                   pl.BlockSpec((_BLK_N, HP), lambda i: (i, 0))],
        out_shape=[jax.ShapeDtypeStruct((NP_, H), jnp.float32),
                   jax.ShapeDtypeStruct((NP_, HP), jnp.float32)],
    )(xp, w_in, wn0p)


def _gate_call(gath, ea, wep, wgext):
    def body(g_ref, ea_ref, we_ref, wg_ref, out_ref):
        a = jnp.dot(ea_ref[...], we_ref[...], preferred_element_type=jnp.float32)
        m = g_ref[...] * a
        z = jnp.dot(m[:, :S0], wg_ref[...], preferred_element_type=jnp.float32)
        res = m * jax.nn.sigmoid(z)
        out_ref[0] = res[:, :128]
        out_ref[1] = res[:, 128:]

    ne = gath.shape[0]
    return pl.pallas_call(
        body,
        grid=(ne // _BLK_E,),
        in_specs=[pl.BlockSpec((_BLK_E, HP), lambda i: (i, 0)),
                  pl.BlockSpec((_BLK_E, 16), lambda i: (i, 0)),
                  pl.BlockSpec((16, HP), lambda i: (0, 0)),
                  pl.BlockSpec((S0, HP), lambda i: (0, 0))],
        out_specs=pl.BlockSpec((2, _BLK_E, 128), lambda i: (0, i, 0)),
        out_shape=jax.ShapeDtypeStruct((2, ne, 128), jnp.float32),
    )(gath, ea, wep, wgext)


def _update_call(h, agg, wlin_msg, wn_upd, wm_upd, wg_ext, wlin_upd, wn_next):
    has_next = wn_next is not None

    def body(h_ref, agg_ref, wl_ref, wn_ref, wm_ref, wg_ref, wu_ref,
             *rest):
        if has_next:
            wnx_ref, h_out, g_out = rest
        else:
            (h_out,) = rest
        h_ = h_ref[...]
        agg = jnp.concatenate([agg_ref[0], agg_ref[1][:, :H - 128]], axis=1)
        aggw = jnp.dot(agg, wl_ref[...],
                       preferred_element_type=jnp.float32)
        u1 = (jnp.dot(h_, wn_ref[...], preferred_element_type=jnp.float32)
              * jnp.dot(aggw, wm_ref[...], preferred_element_type=jnp.float32))
        z = jnp.dot(u1[:, :S0], wg_ref[...], preferred_element_type=jnp.float32)
        u2 = u1 * jax.nn.sigmoid(z)
        hn = h_ + jnp.dot(u2, wu_ref[...], preferred_element_type=jnp.float32)
        h_out[...] = hn
        if has_next:
            g_out[...] = jnp.dot(hn, wnx_ref[...],
                                 preferred_element_type=jnp.float32)

    in_specs = [pl.BlockSpec((_BLK_N, H), lambda i: (i, 0)),
                pl.BlockSpec((2, _BLK_N, 128), lambda i: (0, i, 0)),
                pl.BlockSpec((H, H), lambda i: (0, 0)),
                pl.BlockSpec((H, H), lambda i: (0, 0)),
                pl.BlockSpec((H, H), lambda i: (0, 0)),
                pl.BlockSpec((S0, H), lambda i: (0, 0)),
                pl.BlockSpec((H, H), lambda i: (0, 0))]
    out_specs = [pl.BlockSpec((_BLK_N, H), lambda i: (i, 0))]
    out_shape = [jax.ShapeDtypeStruct((NP_, H), jnp.float32)]
    args = [h, agg, wlin_msg, wn_upd, wm_upd, wg_ext, wlin_upd]
    if has_next:
        in_specs.append(pl.BlockSpec((H, HP), lambda i: (0, 0)))
        out_specs.append(pl.BlockSpec((_BLK_N, HP), lambda i: (i, 0)))
        out_shape.append(jax.ShapeDtypeStruct((NP_, HP), jnp.float32))
        args.append(wn_next)

    return pl.pallas_call(
        body,
        grid=(NP_ // _BLK_N,),
        in_specs=in_specs,
        out_specs=out_specs,
        out_shape=out_shape,
    )(*args)


def _pool_call(h, batch2, w_final):
    nb = NP_ // _BLK_N

    def body(h_ref, b_ref, wf_ref, out_ref, acc_s, cnt_s):
        i = pl.program_id(0)

        @pl.when(i == 0)
        def _():
            acc_s[...] = jnp.zeros_like(acc_s)
            cnt_s[...] = jnp.zeros_like(cnt_s)

        inv = h_ref[:, :S0]
        b = b_ref[...]
        oh = (b == lax.broadcasted_iota(jnp.int32, (_BLK_N, NG), 1)
              ).astype(jnp.float32)
        acc_s[...] += lax.dot_general(oh, inv, (((0,), (0,)), ((), ())),
                                      preferred_element_type=jnp.float32)
        cnt_s[...] += lax.dot_general(oh, jnp.ones((_BLK_N, 8), jnp.float32),
                                      (((0,), (0,)), ((), ())),
                                      preferred_element_type=jnp.float32)

        @pl.when(i == nb - 1)
        def _():
            pooled = acc_s[...] / jnp.maximum(cnt_s[:, :1], 1.0)
            out_ref[...] = jnp.dot(pooled, wf_ref[...],
                                   preferred_element_type=jnp.float32)

    return pl.pallas_call(
        body,
        grid=(nb,),
        in_specs=[pl.BlockSpec((_BLK_N, H), lambda i: (i, 0)),
                  pl.BlockSpec((_BLK_N, 1), lambda i: (i, 0)),
                  pl.BlockSpec((S0, LATENT), lambda i: (0, 0))],
        out_specs=pl.BlockSpec((NG, LATENT), lambda i: (0, 0)),
        out_shape=jax.ShapeDtypeStruct((NG, LATENT), jnp.float32),
        scratch_shapes=[pltpu.VMEM((NG, S0), jnp.float32),
                        pltpu.VMEM((NG, 8), jnp.float32)],
    )(h, batch2, w_final)


@jax.jit
def kernel(x, pos, edge_index, batch, W_in, Wn_msg, We_msg, Wg_msg, Wlin_msg,
           Wn_upd, Wm_upd, Wg_upd, Wlin_upd, W_final):
    f32 = jnp.float32
    row = edge_index[0]
    col = edge_index[1]

    xp = jnp.zeros((NP_, D_IN), f32).at[:N].set(x)
    pos128 = jnp.zeros((NP_, 128), f32).at[:N, :3].set(pos)
    idx_pad = jnp.zeros((80, CHUNK), jnp.int32)
    rowp = jnp.concatenate(
        [row, jnp.full((EP - E,), N, jnp.int32)]).reshape(EP // CHUNK, CHUNK)
    colp = jnp.concatenate(
        [col, jnp.zeros((EP - E,), jnp.int32)]).reshape(EP // CHUNK, CHUNK)
    rowp_g = jnp.concatenate([rowp, idx_pad], axis=0)
    colp_g = jnp.concatenate([colp, idx_pad], axis=0)
    batch2 = jnp.concatenate(
        [batch, jnp.full((NP_ - N,), NG, jnp.int32)]).reshape(NP_, 1)

    eye = jnp.eye(S0, dtype=f32)
    wn_msg_p = jnp.zeros((L, H, HP), f32).at[:, :, :H].set(Wn_msg)
    we_p = jnp.zeros((L, 16, HP), f32).at[:, :4, :H].set(We_msg)
    wg_msg_ext = jnp.zeros((L, S0, HP), f32)
    wg_msg_ext = wg_msg_ext.at[:, :, :S0].set(eye)
    wg_msg_ext = wg_msg_ext.at[:, :, S0:H].set(Wg_msg)
    wg_upd_ext = jnp.concatenate(
        [jnp.broadcast_to(eye, (L, S0, S0)), Wg_upd], axis=2)

    pr, pc = _sc_gather_pos(pos128, rowp_g, colp_g)
    ea = _edge_attr_call(pr, pc)

    h, g = _prologue_call(xp, W_in, wn_msg_p[0])
    for l in range(L):
        gath = _sc_gather256(g, colp_g)
        gated = _gate_call(gath, ea, we_p[l], wg_msg_ext[l])
        agg = _sc_scatter(gated, rowp)
        wn_next = wn_msg_p[l + 1] if l + 1 < L else None
        res = _update_call(h, agg, Wlin_msg[l], Wn_upd[l], Wm_upd[l],
                           wg_upd_ext[l], Wlin_upd[l], wn_next)
        if wn_next is not None:
            h, g = res
        else:
            (h,) = res

    return _pool_call(h, batch2, W_final)
